# baseline (speedup 1.0000x reference)
import functools

import jax
import jax.numpy as jnp
from jax import lax
from jax.experimental import pallas as pl
from jax.experimental.pallas import tpu as pltpu
from jax.experimental.pallas import tpu_sc as plsc

NEG_SLOPE = 0.2
ROWW = 144
HROW = 80
BATCH = 112
NTILES = 32
NBLK = 256


def _tc_prep(xp, W, attmat, NP, HC):
    nq = HC // 128

    def body(x_ref, w_ref, am_ref, h4_ref, ad_ref):
        hb = jnp.dot(x_ref[...], w_ref[...],
                     preferred_element_type=jnp.float32,
                     precision=lax.Precision.HIGHEST)
        ac = jnp.dot(hb, am_ref[...],
                     preferred_element_type=jnp.float32,
                     precision=lax.Precision.HIGHEST)
        def pack_pair(e, o):
            eb = e.astype(jnp.bfloat16).astype(jnp.float32)
            ob = o.astype(jnp.bfloat16).astype(jnp.float32)
            ei = lax.bitcast_convert_type(eb, jnp.int32)
            oi = lax.bitcast_convert_type(ob, jnp.int32)
            return lax.shift_right_logical(ei, 16) | oi

        zpad = jnp.zeros((NBLK, HROW - 65), jnp.int32)
        chunks = []
        for q in range(nq):
            hb4 = hb[:, q * 128:(q + 1) * 128].reshape(NBLK, 4, 2, 16)
            ci = pack_pair(hb4[:, :, 0, :], hb4[:, :, 1, :]).reshape(NBLK, 64)
            ai = pack_pair(ac[:, 2 * q:2 * q + 1], ac[:, 2 * q + 1:2 * q + 2])
            chunks.append(jnp.concatenate([ci, ai, zpad], axis=1))
        h4_ref[...] = jnp.stack(chunks, axis=0)
        ac8 = ac[:, 8:16].reshape(NBLK, 4, 2)
        adi = pack_pair(ac8[:, :, 0], ac8[:, :, 1])
        ad_ref[...] = jnp.concatenate(
            [adi, jnp.zeros((NBLK, 12), jnp.int32)], axis=1)

    return pl.pallas_call(
        body,
        grid=(NP // NBLK,),
        in_specs=[
            pl.BlockSpec((NBLK, xp.shape[1]), lambda i: (i, 0)),
            pl.BlockSpec((xp.shape[1], HC), lambda i: (0, 0)),
            pl.BlockSpec((HC, 16), lambda i: (0, 0)),
        ],
        out_specs=[
            pl.BlockSpec((nq, NBLK, HROW), lambda i: (0, i, 0)),
            pl.BlockSpec((NBLK, 16), lambda i: (i, 0)),
        ],
        out_shape=[
            jax.ShapeDtypeStruct((nq, NP, HROW), jnp.int32),
            jax.ShapeDtypeStruct((NP, 16), jnp.int32),
        ],
    )(xp, W, attmat)


def _bcast_lane(v, l):
    idx = jnp.full((16, 1), l, jnp.int32)
    dn = lax.GatherDimensionNumbers(
        offset_dims=(), collapsed_slice_dims=(0,), start_index_map=(0,))
    return lax.gather(v, idx, dn, slice_sizes=(1,),
                      mode=lax.GatherScatterMode.PROMISE_IN_BOUNDS)


def _sc_edge(h4, acatd, packed3, zeros_hbm, NP, NA, nb):
    nq = h4.shape[0]
    mesh = plsc.VectorSubcoreMesh(core_axis_name="c", subcore_axis_name="s",
                                  num_cores=2, num_subcores=16)
    rows_per_tile = NA // 16

    @functools.partial(
        pl.kernel,
        out_type=jax.ShapeDtypeStruct((nq, 2, NP, ROWW), jnp.float32),
        mesh=mesh,
        scratch_types=[
            pltpu.VMEM_SHARED((NA, ROWW), jnp.float32),
            pltpu.VMEM((nb, BATCH), jnp.int32),
            pltpu.VMEM((BATCH,), jnp.int32),
            pltpu.VMEM((BATCH,), jnp.int32),
            pltpu.VMEM((BATCH, HROW), jnp.int32),
            pltpu.VMEM((BATCH, 16), jnp.int32),
            pltpu.VMEM((BATCH, ROWW), jnp.float32),
            pltpu.SemaphoreType.DMA,
            pltpu.SemaphoreType.DMA,
        ],
        compiler_params=pltpu.CompilerParams(needs_layout_passes=False,
                                             use_tc_tiling_on_sc=False),
    )
    def sc_kernel(h4_hbm, ad_hbm, pk_hbm, z_hbm, out_hbm,
                  acc, idp, isb, idb, hbuf, abuf, sbuf, semh, sema):
        cid = lax.axis_index("c")
        sid = lax.axis_index("s")
        tid = cid * 16 + sid
        lo_mask = jnp.int32(-65536)
        pltpu.sync_copy(pk_hbm.at[tid], idp)

        for q in range(nq):
            pltpu.sync_copy(z_hbm.at[pl.ds(sid * rows_per_tile, rows_per_tile)],
                            acc.at[pl.ds(sid * rows_per_tile, rows_per_tile)])
            plsc.subcore_barrier()

            @pl.loop(0, nb)
            def _batch(j, q=q):
                @pl.loop(0, BATCH // 16)
                def _unpack(g):
                    v = idp[j, pl.ds(g * 16, 16)]
                    isb[pl.ds(g * 16, 16)] = v >> 14
                    idb[pl.ds(g * 16, 16)] = v & 16383

                cp1 = pltpu.async_copy(h4_hbm.at[q].at[isb], hbuf, semh)
                cp2 = pltpu.async_copy(ad_hbm.at[idb], abuf, sema)
                cp1.wait()
                cp2.wait()

                @pl.loop(0, BATCH // 16)
                def _group(g, q=q):
                    i0 = g * 16
                    lane_id = lax.iota(jnp.int32, 16)
                    idx = i0 + lane_id
                    pv = plsc.load_gather(
                        hbuf, [idx, jnp.full((16,), 64, jnp.int32)])
                    av = plsc.load_gather(
                        abuf, [idx, jnp.full((16,), q, jnp.int32)])
                    as0 = plsc.bitcast(pv << 16, jnp.float32)
                    as1 = plsc.bitcast(pv & lo_mask, jnp.float32)
                    ad0 = plsc.bitcast(av << 16, jnp.float32)
                    ad1 = plsc.bitcast(av & lo_mask, jnp.float32)
                    al0 = as0 + ad0
                    al1 = as1 + ad1
                    al0 = jnp.where(al0 > 0, al0, al0 * NEG_SLOPE)
                    al1 = jnp.where(al1 > 0, al1, al1 * NEG_SLOPE)
                    s0 = jnp.exp(al0)
                    s1 = jnp.exp(al1)
                    for l in range(16):
                        b0 = _bcast_lane(s0, l)
                        b1 = _bcast_lane(s1, l)
                        r = i0 + l
                        for k in range(4):
                            sv = b0 if k < 2 else b1
                            w = hbuf[r, pl.ds(k * 16, 16)]
                            lo = plsc.bitcast(w << 16, jnp.float32)
                            hi = plsc.bitcast(w & lo_mask, jnp.float32)
                            sbuf[r, pl.ds(k * 32, 16)] = lo * sv
                            sbuf[r, pl.ds(k * 32 + 16, 16)] = hi * sv
                        tail = jnp.where(
                            lane_id == 0, b0,
                            jnp.where(lane_id == 1, b1,
                                      jnp.zeros((16,), jnp.float32)))
                        sbuf[r, pl.ds(128, 16)] = tail

                pltpu.sync_copy(sbuf, acc.at[idb], add=True)

            plsc.subcore_barrier()
            pltpu.sync_copy(
                acc.at[pl.ds(sid * rows_per_tile, rows_per_tile)],
                out_hbm.at[q, cid, pl.ds(sid * rows_per_tile, rows_per_tile)])
            plsc.subcore_barrier()

    return sc_kernel(h4, acatd, packed3, zeros_hbm)


def _tc_finish(out4, bias2d, NP, H, C):
    nq = out4.shape[0]

    def body(o_ref, b_ref, y_ref):
        a = o_ref[...]
        acc = a[:, 0] + a[:, 1]
        tot = jnp.zeros((NBLK, C), jnp.float32)
        for q in range(nq):
            d0 = acc[q, :, 128:129]
            d1 = acc[q, :, 129:130]
            tot = tot + acc[q, :, 0:C] / d0 + acc[q, :, C:2 * C] / d1
        v = tot * (1.0 / H) + b_ref[...]
        v = jnp.where(v > 0, v, jnp.exp(jnp.minimum(v, 0.0)) - 1.0)
        m = jnp.max(v, axis=-1, keepdims=True)
        z = v - m
        lse = jnp.log(jnp.sum(jnp.exp(z), axis=-1, keepdims=True))
        y_ref[...] = z - lse

    return pl.pallas_call(
        body,
        grid=(NP // NBLK,),
        in_specs=[
            pl.BlockSpec((nq, 2, NBLK, ROWW), lambda i: (0, 0, i, 0)),
            pl.BlockSpec((1, C), lambda i: (0, 0)),
        ],
        out_specs=pl.BlockSpec((NBLK, C), lambda i: (i, 0)),
        out_shape=jax.ShapeDtypeStruct((NP, C), jnp.float32),
    )(out4, bias2d)


def kernel(x, edge_index, W, att_src, att_dst, bias):
    N, NF = x.shape
    HC = W.shape[1]
    H = att_src.shape[1]
    C = att_src.shape[2]
    E = edge_index.shape[1]

    NP = ((N + NBLK - 1) // NBLK) * NBLK
    NA = ((N + 1 + 15) // 16) * 16
    E2 = E + N
    nb = (E2 + NTILES * BATCH - 1) // (NTILES * BATCH)
    EP = nb * NTILES * BATCH

    xp = jnp.pad(x, ((0, NP - N), (0, 0)))

    eye = jnp.eye(H, dtype=jnp.float32)
    am_src = (eye[:, None, :] * att_src[0][:, :, None]).reshape(HC, H)
    am_dst = (eye[:, None, :] * att_dst[0][:, :, None]).reshape(HC, H)
    attmat = jnp.concatenate([am_src, am_dst], axis=1)

    loops = jnp.arange(N, dtype=jnp.int32)
    src = jnp.concatenate([edge_index[0].astype(jnp.int32), loops,
                           jnp.full((EP - E2,), N, jnp.int32)])
    dst = jnp.concatenate([edge_index[1].astype(jnp.int32), loops,
                           jnp.full((EP - E2,), N, jnp.int32)])
    packed3 = (src * 16384 + dst).reshape(NTILES, nb, BATCH)

    zeros_hbm = jnp.zeros((NA, ROWW), jnp.float32)

    h4, acatd = _tc_prep(xp, W, attmat, NP, HC)
    out4 = _sc_edge(h4, acatd, packed3, zeros_hbm, NP, NA, nb)
    y = _tc_finish(out4, bias.reshape(1, C), NP, H, C)
    return y[:N]


# speedup vs baseline: 1.3972x; 1.3972x over previous
import functools

import jax
import jax.numpy as jnp
from jax import lax
from jax.experimental import pallas as pl
from jax.experimental.pallas import tpu as pltpu
from jax.experimental.pallas import tpu_sc as plsc

NEG_SLOPE = 0.2
ROWW = 144
HROW = 80
BATCH = 112
NTILES = 32
NBLK = 256


def _tc_prep(xp, W, attmat, NP, HC):
    nq = HC // 128

    def body(x_ref, w_ref, am_ref, h4_ref, ad_ref):
        hb = jnp.dot(x_ref[...], w_ref[...],
                     preferred_element_type=jnp.float32,
                     precision=lax.Precision.HIGHEST)
        ac = jnp.dot(hb, am_ref[...],
                     preferred_element_type=jnp.float32,
                     precision=lax.Precision.HIGHEST)
        def pack_pair(e, o):
            eb = e.astype(jnp.bfloat16).astype(jnp.float32)
            ob = o.astype(jnp.bfloat16).astype(jnp.float32)
            ei = lax.bitcast_convert_type(eb, jnp.int32)
            oi = lax.bitcast_convert_type(ob, jnp.int32)
            return lax.shift_right_logical(ei, 16) | oi

        zpad = jnp.zeros((NBLK, HROW - 65), jnp.int32)
        chunks = []
        for q in range(nq):
            hb4 = hb[:, q * 128:(q + 1) * 128].reshape(NBLK, 4, 2, 16)
            ci = pack_pair(hb4[:, :, 0, :], hb4[:, :, 1, :]).reshape(NBLK, 64)
            ai = pack_pair(ac[:, 2 * q:2 * q + 1], ac[:, 2 * q + 1:2 * q + 2])
            chunks.append(jnp.concatenate([ci, ai, zpad], axis=1))
        h4_ref[...] = jnp.stack(chunks, axis=0)
        ac8 = ac[:, 8:16].reshape(NBLK, 4, 2)
        adi = pack_pair(ac8[:, :, 0], ac8[:, :, 1])
        ad_ref[...] = jnp.concatenate(
            [adi, jnp.zeros((NBLK, 12), jnp.int32)], axis=1)

    return pl.pallas_call(
        body,
        grid=(NP // NBLK,),
        in_specs=[
            pl.BlockSpec((NBLK, xp.shape[1]), lambda i: (i, 0)),
            pl.BlockSpec((xp.shape[1], HC), lambda i: (0, 0)),
            pl.BlockSpec((HC, 16), lambda i: (0, 0)),
        ],
        out_specs=[
            pl.BlockSpec((nq, NBLK, HROW), lambda i: (0, i, 0)),
            pl.BlockSpec((NBLK, 16), lambda i: (i, 0)),
        ],
        out_shape=[
            jax.ShapeDtypeStruct((nq, NP, HROW), jnp.int32),
            jax.ShapeDtypeStruct((NP, 16), jnp.int32),
        ],
    )(xp, W, attmat)


def _bcast_lane(v, l):
    idx = jnp.full((16, 1), l, jnp.int32)
    dn = lax.GatherDimensionNumbers(
        offset_dims=(), collapsed_slice_dims=(0,), start_index_map=(0,))
    return lax.gather(v, idx, dn, slice_sizes=(1,),
                      mode=lax.GatherScatterMode.PROMISE_IN_BOUNDS)


def _sc_edge(h4, acatd, packed3, zeros_hbm, NP, NA, nb):
    nq = h4.shape[0]
    mesh = plsc.VectorSubcoreMesh(core_axis_name="c", subcore_axis_name="s",
                                  num_cores=2, num_subcores=16)
    rows_per_tile = NA // 16

    @functools.partial(
        pl.kernel,
        out_type=jax.ShapeDtypeStruct((nq, 2, NP, ROWW), jnp.float32),
        mesh=mesh,
        scratch_types=[
            pltpu.VMEM_SHARED((NA, ROWW), jnp.float32),
            pltpu.VMEM((nb, BATCH), jnp.int32),
            pltpu.VMEM((BATCH,), jnp.int32),
            pltpu.VMEM((BATCH,), jnp.int32),
            pltpu.VMEM((BATCH, HROW), jnp.int32),
            pltpu.VMEM((BATCH, 16), jnp.int32),
            pltpu.VMEM((BATCH, ROWW), jnp.float32),
            pltpu.SemaphoreType.DMA,
            pltpu.SemaphoreType.DMA,
        ],
        compiler_params=pltpu.CompilerParams(needs_layout_passes=False,
                                             use_tc_tiling_on_sc=False),
    )
    def sc_kernel(h4_hbm, ad_hbm, pk_hbm, z_hbm, out_hbm,
                  acc, idp, isb, idb, hbuf, abuf, sbuf, semh, sema):
        cid = lax.axis_index("c")
        sid = lax.axis_index("s")
        tid = cid * 16 + sid
        lo_mask = jnp.int32(-65536)
        pltpu.sync_copy(pk_hbm.at[tid], idp)

        @pl.loop(0, nq)
        def _pass(q):
            pltpu.sync_copy(z_hbm.at[pl.ds(sid * rows_per_tile, rows_per_tile)],
                            acc.at[pl.ds(sid * rows_per_tile, rows_per_tile)])
            plsc.subcore_barrier()

            @pl.loop(0, nb)
            def _batch(j, q=q):
                @pl.loop(0, BATCH // 16)
                def _unpack(g):
                    v = idp[j, pl.ds(g * 16, 16)]
                    isb[pl.ds(g * 16, 16)] = v >> 14
                    idb[pl.ds(g * 16, 16)] = v & 16383

                cp1 = pltpu.async_copy(h4_hbm.at[q].at[isb], hbuf, semh)
                cp2 = pltpu.async_copy(ad_hbm.at[idb], abuf, sema)
                cp1.wait()
                cp2.wait()

                for g in range(BATCH // 16):
                    i0 = g * 16
                    lane_id = lax.iota(jnp.int32, 16)
                    idx = i0 + lane_id
                    pv = plsc.load_gather(
                        hbuf, [idx, jnp.full((16,), 64, jnp.int32)])
                    av = plsc.load_gather(
                        abuf, [idx, jnp.full((16,), q, jnp.int32)])
                    as0 = plsc.bitcast(pv << 16, jnp.float32)
                    as1 = plsc.bitcast(pv & lo_mask, jnp.float32)
                    ad0 = plsc.bitcast(av << 16, jnp.float32)
                    ad1 = plsc.bitcast(av & lo_mask, jnp.float32)
                    al0 = as0 + ad0
                    al1 = as1 + ad1
                    al0 = jnp.where(al0 > 0, al0, al0 * NEG_SLOPE)
                    al1 = jnp.where(al1 > 0, al1, al1 * NEG_SLOPE)
                    s0 = jnp.exp(al0)
                    s1 = jnp.exp(al1)
                    for l in range(16):
                        b0 = _bcast_lane(s0, l)
                        b1 = _bcast_lane(s1, l)
                        r = i0 + l
                        for k in range(4):
                            sv = b0 if k < 2 else b1
                            w = hbuf[r, pl.ds(k * 16, 16)]
                            lo = plsc.bitcast(w << 16, jnp.float32)
                            hi = plsc.bitcast(w & lo_mask, jnp.float32)
                            sbuf[r, pl.ds(k * 32, 16)] = lo * sv
                            sbuf[r, pl.ds(k * 32 + 16, 16)] = hi * sv
                        tail = jnp.where(
                            lane_id == 0, b0,
                            jnp.where(lane_id == 1, b1,
                                      jnp.zeros((16,), jnp.float32)))
                        sbuf[r, pl.ds(128, 16)] = tail

                pltpu.sync_copy(sbuf, acc.at[idb], add=True)

            plsc.subcore_barrier()
            pltpu.sync_copy(
                acc.at[pl.ds(sid * rows_per_tile, rows_per_tile)],
                out_hbm.at[q, cid, pl.ds(sid * rows_per_tile, rows_per_tile)])
            plsc.subcore_barrier()

    return sc_kernel(h4, acatd, packed3, zeros_hbm)


def _tc_finish(out4, bias2d, NP, H, C):
    nq = out4.shape[0]

    def body(o_ref, b_ref, y_ref):
        a = o_ref[...]
        acc = a[:, 0] + a[:, 1]
        tot = jnp.zeros((NBLK, C), jnp.float32)
        for q in range(nq):
            d0 = acc[q, :, 128:129]
            d1 = acc[q, :, 129:130]
            tot = tot + acc[q, :, 0:C] / d0 + acc[q, :, C:2 * C] / d1
        v = tot * (1.0 / H) + b_ref[...]
        v = jnp.where(v > 0, v, jnp.exp(jnp.minimum(v, 0.0)) - 1.0)
        m = jnp.max(v, axis=-1, keepdims=True)
        z = v - m
        lse = jnp.log(jnp.sum(jnp.exp(z), axis=-1, keepdims=True))
        y_ref[...] = z - lse

    return pl.pallas_call(
        body,
        grid=(NP // NBLK,),
        in_specs=[
            pl.BlockSpec((nq, 2, NBLK, ROWW), lambda i: (0, 0, i, 0)),
            pl.BlockSpec((1, C), lambda i: (0, 0)),
        ],
        out_specs=pl.BlockSpec((NBLK, C), lambda i: (i, 0)),
        out_shape=jax.ShapeDtypeStruct((NP, C), jnp.float32),
    )(out4, bias2d)


def kernel(x, edge_index, W, att_src, att_dst, bias):
    N, NF = x.shape
    HC = W.shape[1]
    H = att_src.shape[1]
    C = att_src.shape[2]
    E = edge_index.shape[1]

    NP = ((N + NBLK - 1) // NBLK) * NBLK
    NA = ((N + 1 + 15) // 16) * 16
    E2 = E + N
    nb = (E2 + NTILES * BATCH - 1) // (NTILES * BATCH)
    EP = nb * NTILES * BATCH

    xp = jnp.pad(x, ((0, NP - N), (0, 0)))

    eye = jnp.eye(H, dtype=jnp.float32)
    am_src = (eye[:, None, :] * att_src[0][:, :, None]).reshape(HC, H)
    am_dst = (eye[:, None, :] * att_dst[0][:, :, None]).reshape(HC, H)
    attmat = jnp.concatenate([am_src, am_dst], axis=1)

    loops = jnp.arange(N, dtype=jnp.int32)
    src = jnp.concatenate([edge_index[0].astype(jnp.int32), loops,
                           jnp.full((EP - E2,), N, jnp.int32)])
    dst = jnp.concatenate([edge_index[1].astype(jnp.int32), loops,
                           jnp.full((EP - E2,), N, jnp.int32)])
    packed3 = (src * 16384 + dst).reshape(NTILES, nb, BATCH)

    zeros_hbm = jnp.zeros((NA, ROWW), jnp.float32)

    h4, acatd = _tc_prep(xp, W, attmat, NP, HC)
    out4 = _sc_edge(h4, acatd, packed3, zeros_hbm, NP, NA, nb)
    y = _tc_finish(out4, bias.reshape(1, C), NP, H, C)
    return y[:N]


# speedup vs baseline: 1.7796x; 1.2737x over previous
import functools

import jax
import jax.numpy as jnp
from jax import lax
from jax.experimental import pallas as pl
from jax.experimental.pallas import tpu as pltpu
from jax.experimental.pallas import tpu_sc as plsc

NEG_SLOPE = 0.2
ROWW = 144
HROW = 80
BATCH = 80
NTILES = 32
NBLK = 256


def _tc_prep(xp, W, attmat, NP, HC):
    nq = HC // 128

    def body(x_ref, w_ref, am_ref, h4_ref, ad_ref):
        hb = jnp.dot(x_ref[...], w_ref[...],
                     preferred_element_type=jnp.float32,
                     precision=lax.Precision.HIGHEST)
        ac = jnp.dot(hb, am_ref[...],
                     preferred_element_type=jnp.float32,
                     precision=lax.Precision.HIGHEST)
        def pack_pair(e, o):
            eb = e.astype(jnp.bfloat16).astype(jnp.float32)
            ob = o.astype(jnp.bfloat16).astype(jnp.float32)
            ei = lax.bitcast_convert_type(eb, jnp.int32)
            oi = lax.bitcast_convert_type(ob, jnp.int32)
            return lax.shift_right_logical(ei, 16) | oi

        zpad = jnp.zeros((NBLK, HROW - 65), jnp.int32)
        chunks = []
        for q in range(nq):
            hb4 = hb[:, q * 128:(q + 1) * 128].reshape(NBLK, 4, 2, 16)
            ci = pack_pair(hb4[:, :, 0, :], hb4[:, :, 1, :]).reshape(NBLK, 64)
            ai = pack_pair(ac[:, 2 * q:2 * q + 1], ac[:, 2 * q + 1:2 * q + 2])
            chunks.append(jnp.concatenate([ci, ai, zpad], axis=1))
        h4_ref[...] = jnp.stack(chunks, axis=0)
        ac8 = ac[:, 8:16].reshape(NBLK, 4, 2)
        adi = pack_pair(ac8[:, :, 0], ac8[:, :, 1])
        ad_ref[...] = jnp.concatenate(
            [adi, jnp.zeros((NBLK, 12), jnp.int32)], axis=1)

    return pl.pallas_call(
        body,
        grid=(NP // NBLK,),
        in_specs=[
            pl.BlockSpec((NBLK, xp.shape[1]), lambda i: (i, 0)),
            pl.BlockSpec((xp.shape[1], HC), lambda i: (0, 0)),
            pl.BlockSpec((HC, 16), lambda i: (0, 0)),
        ],
        out_specs=[
            pl.BlockSpec((nq, NBLK, HROW), lambda i: (0, i, 0)),
            pl.BlockSpec((NBLK, 16), lambda i: (i, 0)),
        ],
        out_shape=[
            jax.ShapeDtypeStruct((nq, NP, HROW), jnp.int32),
            jax.ShapeDtypeStruct((NP, 16), jnp.int32),
        ],
    )(xp, W, attmat)


def _bcast_lane(v, l):
    idx = jnp.full((16, 1), l, jnp.int32)
    dn = lax.GatherDimensionNumbers(
        offset_dims=(), collapsed_slice_dims=(0,), start_index_map=(0,))
    return lax.gather(v, idx, dn, slice_sizes=(1,),
                      mode=lax.GatherScatterMode.PROMISE_IN_BOUNDS)


def _sc_edge(h4, acatd, packed3, zeros_hbm, NP, NA, nb):
    nq = h4.shape[0]
    mesh = plsc.VectorSubcoreMesh(core_axis_name="c", subcore_axis_name="s",
                                  num_cores=2, num_subcores=16)
    rows_per_tile = NA // 16

    @functools.partial(
        pl.kernel,
        out_type=jax.ShapeDtypeStruct((nq, 2, NP, ROWW), jnp.float32),
        mesh=mesh,
        scratch_types=[
            pltpu.VMEM_SHARED((NA, ROWW), jnp.float32),
            pltpu.VMEM((nb, BATCH), jnp.int32),
            pltpu.VMEM((2, BATCH), jnp.int32),
            pltpu.VMEM((2, BATCH), jnp.int32),
            pltpu.VMEM((BATCH, HROW), jnp.int32),
            pltpu.VMEM((BATCH, HROW), jnp.int32),
            pltpu.VMEM((BATCH, 16), jnp.int32),
            pltpu.VMEM((BATCH, 16), jnp.int32),
            pltpu.VMEM((BATCH, ROWW), jnp.float32),
            pltpu.SemaphoreType.DMA,
            pltpu.SemaphoreType.DMA,
            pltpu.SemaphoreType.DMA,
            pltpu.SemaphoreType.DMA,
        ],
        compiler_params=pltpu.CompilerParams(needs_layout_passes=False,
                                             use_tc_tiling_on_sc=False),
    )
    def sc_kernel(h4_hbm, ad_hbm, pk_hbm, z_hbm, out_hbm,
                  acc, idp, isb2, idb2, hbufA, hbufB, abufA, abufB, sbuf,
                  semhA, semhB, semaA, semaB):
        cid = lax.axis_index("c")
        sid = lax.axis_index("s")
        tid = cid * 16 + sid
        lo_mask = jnp.int32(-65536)
        hbufs = (hbufA, hbufB)
        abufs = (abufA, abufB)
        semh = (semhA, semhB)
        sema = (semaA, semaB)
        pltpu.sync_copy(pk_hbm.at[tid], idp)

        def unpack(j, slot):
            @pl.loop(0, BATCH // 16)
            def _unpack(g, slot=slot):
                v = idp[j, pl.ds(g * 16, 16)]
                isb2[slot, pl.ds(g * 16, 16)] = v >> 14
                idb2[slot, pl.ds(g * 16, 16)] = v & 16383

        def issue_gathers(q, slot):
            pltpu.async_copy(h4_hbm.at[q].at[isb2.at[slot]],
                             hbufs[slot], semh[slot])
            pltpu.async_copy(ad_hbm.at[idb2.at[slot]],
                             abufs[slot], sema[slot])

        def drain_gathers(q, slot):
            pltpu.make_async_copy(h4_hbm.at[q].at[isb2.at[slot]],
                                  hbufs[slot], semh[slot]).wait()
            pltpu.make_async_copy(ad_hbm.at[idb2.at[slot]],
                                  abufs[slot], sema[slot]).wait()

        def compute_scatter(q, slot):
            hbuf = hbufs[slot]
            abuf = abufs[slot]
            for g in range(BATCH // 16):
                i0 = g * 16
                lane_id = lax.iota(jnp.int32, 16)
                idx = i0 + lane_id
                pv = plsc.load_gather(
                    hbuf, [idx, jnp.full((16,), 64, jnp.int32)])
                av = plsc.load_gather(
                    abuf, [idx, jnp.full((16,), q, jnp.int32)])
                as0 = plsc.bitcast(pv << 16, jnp.float32)
                as1 = plsc.bitcast(pv & lo_mask, jnp.float32)
                ad0 = plsc.bitcast(av << 16, jnp.float32)
                ad1 = plsc.bitcast(av & lo_mask, jnp.float32)
                al0 = as0 + ad0
                al1 = as1 + ad1
                al0 = jnp.where(al0 > 0, al0, al0 * NEG_SLOPE)
                al1 = jnp.where(al1 > 0, al1, al1 * NEG_SLOPE)
                s0 = jnp.exp(al0)
                s1 = jnp.exp(al1)
                for l in range(16):
                    b0 = _bcast_lane(s0, l)
                    b1 = _bcast_lane(s1, l)
                    r = i0 + l
                    for k in range(4):
                        sv = b0 if k < 2 else b1
                        w = hbuf[r, pl.ds(k * 16, 16)]
                        lo = plsc.bitcast(w << 16, jnp.float32)
                        hi = plsc.bitcast(w & lo_mask, jnp.float32)
                        sbuf[r, pl.ds(k * 32, 16)] = lo * sv
                        sbuf[r, pl.ds(k * 32 + 16, 16)] = hi * sv
                    tail = jnp.where(
                        lane_id == 0, b0,
                        jnp.where(lane_id == 1, b1,
                                  jnp.zeros((16,), jnp.float32)))
                    sbuf[r, pl.ds(128, 16)] = tail
            pltpu.sync_copy(sbuf, acc.at[idb2.at[slot]], add=True)

        @pl.loop(0, nq)
        def _pass(q):
            pltpu.sync_copy(z_hbm.at[pl.ds(sid * rows_per_tile, rows_per_tile)],
                            acc.at[pl.ds(sid * rows_per_tile, rows_per_tile)])
            plsc.subcore_barrier()

            unpack(0, 0)
            issue_gathers(q, 0)

            @pl.loop(0, nb // 2)
            def _pair(jj, q=q):
                j0 = jj * 2
                unpack(j0 + 1, 1)
                issue_gathers(q, 1)
                drain_gathers(q, 0)
                compute_scatter(q, 0)
                @pl.when(jj + 1 < nb // 2)
                def _():
                    unpack(j0 + 2, 0)
                    issue_gathers(q, 0)
                drain_gathers(q, 1)
                compute_scatter(q, 1)

            plsc.subcore_barrier()
            pltpu.sync_copy(
                acc.at[pl.ds(sid * rows_per_tile, rows_per_tile)],
                out_hbm.at[q, cid, pl.ds(sid * rows_per_tile, rows_per_tile)])
            plsc.subcore_barrier()

    return sc_kernel(h4, acatd, packed3, zeros_hbm)


def _tc_finish(out4, bias2d, NP, H, C):
    nq = out4.shape[0]

    def body(o_ref, b_ref, y_ref):
        a = o_ref[...]
        acc = a[:, 0] + a[:, 1]
        tot = jnp.zeros((NBLK, C), jnp.float32)
        for q in range(nq):
            d0 = acc[q, :, 128:129]
            d1 = acc[q, :, 129:130]
            tot = tot + acc[q, :, 0:C] / d0 + acc[q, :, C:2 * C] / d1
        v = tot * (1.0 / H) + b_ref[...]
        v = jnp.where(v > 0, v, jnp.exp(jnp.minimum(v, 0.0)) - 1.0)
        m = jnp.max(v, axis=-1, keepdims=True)
        z = v - m
        lse = jnp.log(jnp.sum(jnp.exp(z), axis=-1, keepdims=True))
        y_ref[...] = z - lse

    return pl.pallas_call(
        body,
        grid=(NP // NBLK,),
        in_specs=[
            pl.BlockSpec((nq, 2, NBLK, ROWW), lambda i: (0, 0, i, 0)),
            pl.BlockSpec((1, C), lambda i: (0, 0)),
        ],
        out_specs=pl.BlockSpec((NBLK, C), lambda i: (i, 0)),
        out_shape=jax.ShapeDtypeStruct((NP, C), jnp.float32),
    )(out4, bias2d)


def kernel(x, edge_index, W, att_src, att_dst, bias):
    N, NF = x.shape
    HC = W.shape[1]
    H = att_src.shape[1]
    C = att_src.shape[2]
    E = edge_index.shape[1]

    NP = ((N + NBLK - 1) // NBLK) * NBLK
    NA = ((N + 1 + 15) // 16) * 16
    E2 = E + N
    nb = (E2 + NTILES * BATCH - 1) // (NTILES * BATCH)
    nb = nb + (nb % 2)
    EP = nb * NTILES * BATCH

    xp = jnp.pad(x, ((0, NP - N), (0, 0)))

    eye = jnp.eye(H, dtype=jnp.float32)
    am_src = (eye[:, None, :] * att_src[0][:, :, None]).reshape(HC, H)
    am_dst = (eye[:, None, :] * att_dst[0][:, :, None]).reshape(HC, H)
    attmat = jnp.concatenate([am_src, am_dst], axis=1)

    loops = jnp.arange(N, dtype=jnp.int32)
    src = jnp.concatenate([edge_index[0].astype(jnp.int32), loops,
                           jnp.full((EP - E2,), N, jnp.int32)])
    dst = jnp.concatenate([edge_index[1].astype(jnp.int32), loops,
                           jnp.full((EP - E2,), N, jnp.int32)])
    packed3 = (src * 16384 + dst).reshape(NTILES, nb, BATCH)

    zeros_hbm = jnp.zeros((NA, ROWW), jnp.float32)

    h4, acatd = _tc_prep(xp, W, attmat, NP, HC)
    out4 = _sc_edge(h4, acatd, packed3, zeros_hbm, NP, NA, nb)
    y = _tc_finish(out4, bias.reshape(1, C), NP, H, C)
    return y[:N]
